# Initial kernel scaffold; baseline (speedup 1.0000x reference)
import jax
import jax.numpy as jnp
from jax.experimental import pallas as pl


def kernel(features, W0, b0, W1, b1):
    raise NotImplementedError("write your pallas kernel here")


# speedup vs baseline: 5.6565x; 5.6565x over previous
import functools

import jax
import jax.numpy as jnp
from jax.experimental import pallas as pl

_K = 30
_LANES = 128


def _emb_kernel(n_valid, x_ref, w0_ref, b0_ref, w1_ref, b1_ref, emb_ref):
    x = x_ref[...]
    h = jax.lax.dot_general(x, w0_ref[...], (((1,), (1,)), ((), ())),
                            preferred_element_type=jnp.float32)
    h = h + b0_ref[...]
    h = jnp.maximum(h, 0.0)
    h = jax.lax.dot_general(h, w1_ref[...], (((1,), (1,)), ((), ())),
                            preferred_element_type=jnp.float32)
    h = h + b1_ref[...]
    nrm = jnp.maximum(jnp.sqrt(jnp.sum(h * h, axis=1, keepdims=True)), 1e-12)
    row = jax.lax.broadcasted_iota(jnp.int32, h.shape, 0)
    emb_ref[...] = jnp.where(row < n_valid, h / nrm, 0.0)


def _sim_kernel(n_valid, embr_ref, emba_ref, out_ref):
    er = embr_ref[...]
    ea = emba_ref[...]
    sim = jax.lax.dot_general(er, ea, (((1,), (1,)), ((), ())),
                              preferred_element_type=jnp.float32)

    def body(_, s):
        m = jnp.max(s, axis=1, keepdims=True)
        return jnp.where(s == m, -3.0, s)

    s = jax.lax.fori_loop(0, _K, body, sim)
    thr = jnp.max(s, axis=1, keepdims=True)

    keep = sim[:, :n_valid]
    out_ref[...] = jnp.where(keep >= thr, jnp.maximum(keep, 0.0), 0.0)


def kernel(features, W0, b0, W1, b1):
    n, d = features.shape
    np_ = ((n + _LANES - 1) // _LANES) * _LANES
    rb = 200 if n % 200 == 0 else n

    fpad = jnp.pad(features, ((0, np_ - n), (0, 0)))

    emb = pl.pallas_call(
        functools.partial(_emb_kernel, n),
        out_shape=jax.ShapeDtypeStruct((np_, d), jnp.float32),
    )(fpad, W0, b0.reshape(1, d), W1, b1.reshape(1, d))

    out = pl.pallas_call(
        functools.partial(_sim_kernel, n),
        grid=(n // rb,),
        in_specs=[
            pl.BlockSpec((rb, d), lambda i: (i, 0)),
            pl.BlockSpec((np_, d), lambda i: (0, 0)),
        ],
        out_specs=pl.BlockSpec((rb, n), lambda i: (i, 0)),
        out_shape=jax.ShapeDtypeStruct((n, n), jnp.float32),
    )(emb, emb)
    return out


# speedup vs baseline: 13.9484x; 2.4659x over previous
import functools

import jax
import jax.numpy as jnp
from jax.experimental import pallas as pl

_K = 30
_LANES = 128
_BISECT = 20


def _emb_kernel(n_valid, x_ref, w0_ref, b0_ref, w1_ref, b1_ref, emb_ref):
    x = x_ref[...]
    h = jax.lax.dot_general(x, w0_ref[...], (((1,), (1,)), ((), ())),
                            preferred_element_type=jnp.float32)
    h = h + b0_ref[...]
    h = jnp.maximum(h, 0.0)
    h = jax.lax.dot_general(h, w1_ref[...], (((1,), (1,)), ((), ())),
                            preferred_element_type=jnp.float32)
    h = h + b1_ref[...]
    nrm = jnp.maximum(jnp.sqrt(jnp.sum(h * h, axis=1, keepdims=True)), 1e-12)
    row = jax.lax.broadcasted_iota(jnp.int32, h.shape, 0)
    emb_ref[...] = jnp.where(row < n_valid, h / nrm, 0.0)


def _sim_kernel(n_valid, embr_ref, emba_ref, out_ref):
    er = embr_ref[...]
    ea = emba_ref[...]
    sim = jax.lax.dot_general(er, ea, (((1,), (1,)), ((), ())),
                              preferred_element_type=jnp.float32)
    rb, np_ = sim.shape
    ngrp = np_ // _LANES

    lane_max = jnp.max(sim.reshape(rb, ngrp, _LANES), axis=1)

    def knock(_, m):
        mx = jnp.max(m, axis=1, keepdims=True)
        return jnp.where(m == mx, -3.0, m)

    t_hi = jnp.max(lane_max, axis=1, keepdims=True) + 1e-5
    t_lo = jnp.max(jax.lax.fori_loop(0, _K, knock, lane_max),
                   axis=1, keepdims=True)

    def bisect(_, carry):
        lo, hi = carry
        tm = 0.5 * (lo + hi)
        cnt = jnp.sum(jnp.where(sim >= tm, 1.0, 0.0), axis=1, keepdims=True)
        ge = cnt >= (_K + 1)
        return jnp.where(ge, tm, lo), jnp.where(ge, hi, tm)

    t_lo, _ = jax.lax.fori_loop(0, _BISECT, bisect, (t_lo, t_hi))

    keep = sim[:, :n_valid]
    out_ref[...] = jnp.where(keep >= t_lo, jnp.maximum(keep, 0.0), 0.0)


def kernel(features, W0, b0, W1, b1):
    n, d = features.shape
    np_ = ((n + _LANES - 1) // _LANES) * _LANES
    rb = 200 if n % 200 == 0 else n

    fpad = jnp.pad(features, ((0, np_ - n), (0, 0)))

    emb = pl.pallas_call(
        functools.partial(_emb_kernel, n),
        out_shape=jax.ShapeDtypeStruct((np_, d), jnp.float32),
    )(fpad, W0, b0.reshape(1, d), W1, b1.reshape(1, d))

    out = pl.pallas_call(
        functools.partial(_sim_kernel, n),
        grid=(n // rb,),
        in_specs=[
            pl.BlockSpec((rb, d), lambda i: (i, 0)),
            pl.BlockSpec((np_, d), lambda i: (0, 0)),
        ],
        out_specs=pl.BlockSpec((rb, n), lambda i: (i, 0)),
        out_shape=jax.ShapeDtypeStruct((n, n), jnp.float32),
    )(emb, emb)
    return out
